# baseline (speedup 1.0000x reference)
import jax
import jax.numpy as jnp
from jax import lax
from jax.experimental import pallas as pl
from jax.experimental.pallas import tpu as pltpu
from jax.experimental.pallas import tpu_sc as plsc

H = 64
B = 16384
NC, NS = 2, 16
NW = NC * NS
BPW = B // NW
LANES = 16

MLP_BLOCK = 2048


def _fetch(tT_hbm, stage, sem, i):
    tb = pl.multiple_of(i - lax.bitwise_and(i, 127), 128)
    pltpu.make_async_copy(tT_hbm.at[:, pl.ds(tb, 128)], stage, sem).start()


def _drain(tT_hbm, stage, sem):
    pltpu.make_async_copy(tT_hbm.at[:, pl.ds(0, 128)], stage, sem).wait()


def _extract(stage, rows_f, i, k):
    lane = jnp.full((LANES,), lax.bitwise_and(i, 127), jnp.int32)
    for m in range(H // LANES):
        j = lax.iota(jnp.int32, LANES) + (m * LANES)
        vals = plsc.load_gather(stage, [j, lane])
        rows_f[pl.ds(k * H + m * LANES, LANES)] = vals


def _sget(idx_v, k):
    base16 = (k // LANES) * LANES
    v = idx_v[pl.ds(base16, LANES)]
    m = lax.iota(jnp.int32, LANES) == (k - base16)
    return jnp.sum(jnp.where(m, v, 0))


def _gather_body(idx_hbm, tT_hbm, out_hbm, idx_v, rows_f,
                 stage0, stage1, sem0, sem1):
    wid = lax.axis_index("s") * NC + lax.axis_index("c")
    base = wid * BPW
    pltpu.sync_copy(idx_hbm.at[pl.ds(base, BPW)], idx_v)

    stages = (stage0, stage1)
    sems = (sem0, sem1)
    _fetch(tT_hbm, stage0, sem0, _sget(idx_v, 0))
    _fetch(tT_hbm, stage1, sem1, _sget(idx_v, 1))

    def body(g, carry):
        for b in range(2):
            k = g * 2 + b
            i = _sget(idx_v, k)
            _drain(tT_hbm, stages[b], sems[b])
            _extract(stages[b], rows_f, i, k)
            _fetch(tT_hbm, stages[b], sems[b], _sget(idx_v, k + 2))
        return carry

    lax.fori_loop(0, BPW // 2 - 1, body, 0)
    for b in range(2):
        k = BPW - 2 + b
        i = _sget(idx_v, k)
        _drain(tT_hbm, stages[b], sems[b])
        _extract(stages[b], rows_f, i, k)

    pltpu.sync_copy(rows_f, out_hbm.at[pl.ds(base * H, BPW * H)])


def _sc_gather(idx, tableT):
    mesh = plsc.VectorSubcoreMesh(core_axis_name="c", subcore_axis_name="s")
    f = pl.kernel(
        _gather_body,
        out_type=jax.ShapeDtypeStruct((B * H,), jnp.float32),
        mesh=mesh,
        scratch_types=[
            pltpu.VMEM((BPW,), jnp.int32),
            pltpu.VMEM((BPW * H,), jnp.float32),
            pltpu.VMEM((H, 128), jnp.float32),
            pltpu.VMEM((H, 128), jnp.float32),
            pltpu.SemaphoreType.DMA,
            pltpu.SemaphoreType.DMA,
        ],
        compiler_params=pltpu.CompilerParams(
            use_tc_tiling_on_sc=True, needs_layout_passes=False),
    )
    return f(idx, tableT)


def _mlp_body(e_ref, w1_ref, b1_ref, w2t_ref, b2_ref, out_ref):
    h = jnp.dot(e_ref[...], w1_ref[...], preferred_element_type=jnp.float32)
    h = jnp.maximum(h + b1_ref[...], 0.0)
    y = jnp.sum(h * w2t_ref[...], axis=1, keepdims=True) + b2_ref[0, 0]
    out_ref[...] = y


def _mlp(e, W1, b1, W2, b2):
    return pl.pallas_call(
        _mlp_body,
        grid=(B // MLP_BLOCK,),
        in_specs=[
            pl.BlockSpec((MLP_BLOCK, H), lambda i: (i, 0)),
            pl.BlockSpec((H, H), lambda i: (0, 0)),
            pl.BlockSpec((1, H), lambda i: (0, 0)),
            pl.BlockSpec((1, H), lambda i: (0, 0)),
            pl.BlockSpec((1, 1), lambda i: (0, 0)),
        ],
        out_specs=pl.BlockSpec((MLP_BLOCK, 1), lambda i: (i, 0)),
        out_shape=jax.ShapeDtypeStruct((B, 1), jnp.float32),
    )(e, W1, b1.reshape(1, H), W2.reshape(1, H), b2.reshape(1, 1))


def kernel(users, items, table, W1, b1, W2, b2):
    idx = users.astype(jnp.int32)
    flat = _sc_gather(idx, table.T)
    e = flat.reshape(B, H)
    return _mlp(e, W1, b1, W2, b2)


# speedup vs baseline: 1.3330x; 1.3330x over previous
import jax
import jax.numpy as jnp
from jax import lax
from jax.experimental import pallas as pl
from jax.experimental.pallas import tpu as pltpu
from jax.experimental.pallas import tpu_sc as plsc

H = 64
B = 16384
NC, NS = 2, 16
NW = NC * NS
BPW = B // NW
LANES = 16

MLP_BLOCK = 2048


def _fetch(tT_hbm, stage, sem, i):
    tb = pl.multiple_of(i - lax.bitwise_and(i, 127), 128)
    pltpu.make_async_copy(tT_hbm.at[:, pl.ds(tb, 128)], stage, sem).start()


def _drain(tT_hbm, stage, sem):
    pltpu.make_async_copy(tT_hbm.at[:, pl.ds(0, 128)], stage, sem).wait()


def _extract(stage, rows_f, i, k):
    lane = jnp.full((LANES,), lax.bitwise_and(i, 127), jnp.int32)
    for m in range(H // LANES):
        j = lax.iota(jnp.int32, LANES) + (m * LANES)
        vals = plsc.load_gather(stage, [j, lane])
        rows_f[pl.ds(k * H + m * LANES, LANES)] = vals


def _sget(idx_v, k):
    base16 = (k // LANES) * LANES
    v = idx_v[pl.ds(base16, LANES)]
    m = lax.iota(jnp.int32, LANES) == (k - base16)
    return jnp.sum(jnp.where(m, v, 0))


NBUF = 4


def _gather_body(idx_hbm, tT_hbm, out_hbm, idx_v, rows_f,
                 stage0, stage1, stage2, stage3, sem0, sem1, sem2, sem3):
    wid = lax.axis_index("s") * NC + lax.axis_index("c")
    base = wid * BPW
    pltpu.sync_copy(idx_hbm.at[pl.ds(base, BPW)], idx_v)

    stages = (stage0, stage1, stage2, stage3)
    sems = (sem0, sem1, sem2, sem3)
    for b in range(NBUF):
        _fetch(tT_hbm, stages[b], sems[b], _sget(idx_v, b))

    def body(g, carry):
        for b in range(NBUF):
            k = g * NBUF + b
            i = _sget(idx_v, k)
            _drain(tT_hbm, stages[b], sems[b])
            _extract(stages[b], rows_f, i, k)
            _fetch(tT_hbm, stages[b], sems[b], _sget(idx_v, k + NBUF))
        return carry

    lax.fori_loop(0, BPW // NBUF - 1, body, 0)
    for b in range(NBUF):
        k = BPW - NBUF + b
        i = _sget(idx_v, k)
        _drain(tT_hbm, stages[b], sems[b])
        _extract(stages[b], rows_f, i, k)

    pltpu.sync_copy(rows_f, out_hbm.at[pl.ds(base * H, BPW * H)])


def _sc_gather(idx, tableT):
    mesh = plsc.VectorSubcoreMesh(core_axis_name="c", subcore_axis_name="s")
    f = pl.kernel(
        _gather_body,
        out_type=jax.ShapeDtypeStruct((B * H,), jnp.float32),
        mesh=mesh,
        scratch_types=[
            pltpu.VMEM((BPW,), jnp.int32),
            pltpu.VMEM((BPW * H,), jnp.float32),
            pltpu.VMEM((H, 128), jnp.float32),
            pltpu.VMEM((H, 128), jnp.float32),
            pltpu.VMEM((H, 128), jnp.float32),
            pltpu.VMEM((H, 128), jnp.float32),
            pltpu.SemaphoreType.DMA,
            pltpu.SemaphoreType.DMA,
            pltpu.SemaphoreType.DMA,
            pltpu.SemaphoreType.DMA,
        ],
        compiler_params=pltpu.CompilerParams(
            use_tc_tiling_on_sc=True, needs_layout_passes=False),
    )
    return f(idx, tableT)


def _mlp_body(e_ref, w1_ref, b1_ref, w2t_ref, b2_ref, out_ref):
    h = jnp.dot(e_ref[...], w1_ref[...], preferred_element_type=jnp.float32)
    h = jnp.maximum(h + b1_ref[...], 0.0)
    y = jnp.sum(h * w2t_ref[...], axis=1, keepdims=True) + b2_ref[0, 0]
    out_ref[...] = y


def _mlp(e, W1, b1, W2, b2):
    return pl.pallas_call(
        _mlp_body,
        grid=(B // MLP_BLOCK,),
        in_specs=[
            pl.BlockSpec((MLP_BLOCK, H), lambda i: (i, 0)),
            pl.BlockSpec((H, H), lambda i: (0, 0)),
            pl.BlockSpec((1, H), lambda i: (0, 0)),
            pl.BlockSpec((1, H), lambda i: (0, 0)),
            pl.BlockSpec((1, 1), lambda i: (0, 0)),
        ],
        out_specs=pl.BlockSpec((MLP_BLOCK, 1), lambda i: (i, 0)),
        out_shape=jax.ShapeDtypeStruct((B, 1), jnp.float32),
    )(e, W1, b1.reshape(1, H), W2.reshape(1, H), b2.reshape(1, 1))


def kernel(users, items, table, W1, b1, W2, b2):
    idx = users.astype(jnp.int32)
    flat = _sc_gather(idx, table.T)
    e = flat.reshape(B, H)
    return _mlp(e, W1, b1, W2, b2)


# speedup vs baseline: 1.5391x; 1.1546x over previous
import jax
import jax.numpy as jnp
from jax import lax
from jax.experimental import pallas as pl
from jax.experimental.pallas import tpu as pltpu
from jax.experimental.pallas import tpu_sc as plsc

H = 64
B = 16384
NC, NS = 2, 16
NW = NC * NS
BPW = B // NW
LANES = 16

MLP_BLOCK = 2048


def _fetch(tT_hbm, stage, sem, i):
    tb = pl.multiple_of(i - lax.bitwise_and(i, 127), 128)
    pltpu.make_async_copy(tT_hbm.at[:, pl.ds(tb, 128)], stage, sem).start()


def _drain(tT_hbm, stage, sem):
    pltpu.make_async_copy(tT_hbm.at[:, pl.ds(0, 128)], stage, sem).wait()


def _extract(stage, rows_f, i, k):
    lane = jnp.full((LANES,), lax.bitwise_and(i, 127), jnp.int32)
    for m in range(H // LANES):
        j = lax.iota(jnp.int32, LANES) + (m * LANES)
        vals = plsc.load_gather(stage, [j, lane])
        rows_f[pl.ds(k * H + m * LANES, LANES)] = vals


def _sget(idx_v, k):
    base16 = (k // LANES) * LANES
    v = idx_v[pl.ds(base16, LANES)]
    m = lax.iota(jnp.int32, LANES) == (k - base16)
    return jnp.sum(jnp.where(m, v, 0))


NBUF = 8


def _gather_body(idx_hbm, tT_hbm, out_hbm, idx_v, rows_f,
                 stage0, stage1, stage2, stage3, stage4, stage5, stage6, stage7,
                 sem0, sem1, sem2, sem3, sem4, sem5, sem6, sem7):
    wid = lax.axis_index("s") * NC + lax.axis_index("c")
    base = wid * BPW
    pltpu.sync_copy(idx_hbm.at[pl.ds(base, BPW)], idx_v)

    stages = (stage0, stage1, stage2, stage3, stage4, stage5, stage6, stage7)
    sems = (sem0, sem1, sem2, sem3, sem4, sem5, sem6, sem7)
    for b in range(NBUF):
        _fetch(tT_hbm, stages[b], sems[b], _sget(idx_v, b))

    def body(g, carry):
        for b in range(NBUF):
            k = g * NBUF + b
            i = _sget(idx_v, k)
            _drain(tT_hbm, stages[b], sems[b])
            _extract(stages[b], rows_f, i, k)
            _fetch(tT_hbm, stages[b], sems[b], _sget(idx_v, k + NBUF))
        return carry

    lax.fori_loop(0, BPW // NBUF - 1, body, 0)
    for b in range(NBUF):
        k = BPW - NBUF + b
        i = _sget(idx_v, k)
        _drain(tT_hbm, stages[b], sems[b])
        _extract(stages[b], rows_f, i, k)

    pltpu.sync_copy(rows_f, out_hbm.at[pl.ds(base * H, BPW * H)])


def _sc_gather(idx, tableT):
    mesh = plsc.VectorSubcoreMesh(core_axis_name="c", subcore_axis_name="s")
    f = pl.kernel(
        _gather_body,
        out_type=jax.ShapeDtypeStruct((B * H,), jnp.float32),
        mesh=mesh,
        scratch_types=[
            pltpu.VMEM((BPW,), jnp.int32),
            pltpu.VMEM((BPW * H,), jnp.float32),
            pltpu.VMEM((H, 128), jnp.float32),
            pltpu.VMEM((H, 128), jnp.float32),
            pltpu.VMEM((H, 128), jnp.float32),
            pltpu.VMEM((H, 128), jnp.float32),
            pltpu.VMEM((H, 128), jnp.float32),
            pltpu.VMEM((H, 128), jnp.float32),
            pltpu.VMEM((H, 128), jnp.float32),
            pltpu.VMEM((H, 128), jnp.float32),
            pltpu.SemaphoreType.DMA,
            pltpu.SemaphoreType.DMA,
            pltpu.SemaphoreType.DMA,
            pltpu.SemaphoreType.DMA,
            pltpu.SemaphoreType.DMA,
            pltpu.SemaphoreType.DMA,
            pltpu.SemaphoreType.DMA,
            pltpu.SemaphoreType.DMA,
        ],
        compiler_params=pltpu.CompilerParams(
            use_tc_tiling_on_sc=True, needs_layout_passes=False),
    )
    return f(idx, tableT)


def _mlp_body(e_ref, w1_ref, b1_ref, w2t_ref, b2_ref, out_ref):
    h = jnp.dot(e_ref[...], w1_ref[...], preferred_element_type=jnp.float32)
    h = jnp.maximum(h + b1_ref[...], 0.0)
    y = jnp.sum(h * w2t_ref[...], axis=1, keepdims=True) + b2_ref[0, 0]
    out_ref[...] = y


def _mlp(e, W1, b1, W2, b2):
    return pl.pallas_call(
        _mlp_body,
        grid=(B // MLP_BLOCK,),
        in_specs=[
            pl.BlockSpec((MLP_BLOCK, H), lambda i: (i, 0)),
            pl.BlockSpec((H, H), lambda i: (0, 0)),
            pl.BlockSpec((1, H), lambda i: (0, 0)),
            pl.BlockSpec((1, H), lambda i: (0, 0)),
            pl.BlockSpec((1, 1), lambda i: (0, 0)),
        ],
        out_specs=pl.BlockSpec((MLP_BLOCK, 1), lambda i: (i, 0)),
        out_shape=jax.ShapeDtypeStruct((B, 1), jnp.float32),
    )(e, W1, b1.reshape(1, H), W2.reshape(1, H), b2.reshape(1, 1))


def kernel(users, items, table, W1, b1, W2, b2):
    idx = users.astype(jnp.int32)
    flat = _sc_gather(idx, table.T)
    e = flat.reshape(B, H)
    return _mlp(e, W1, b1, W2, b2)


# speedup vs baseline: 1.5469x; 1.0051x over previous
import jax
import jax.numpy as jnp
from jax import lax
from jax.experimental import pallas as pl
from jax.experimental.pallas import tpu as pltpu
from jax.experimental.pallas import tpu_sc as plsc

H = 64
B = 16384
NC, NS = 2, 16
NW = NC * NS
BPW = B // NW
LANES = 16

MLP_BLOCK = 2048


def _fetch(tT_hbm, stage, sem, i):
    tb = pl.multiple_of(i - lax.bitwise_and(i, 127), 128)
    for g in range(H // 8):
        pltpu.make_async_copy(
            tT_hbm.at[pl.ds(g * 8, 8), pl.ds(tb, 128)],
            stage.at[pl.ds(g * 8, 8), :],
            sem,
        ).start()


def _drain(tT_hbm, stage, sem):
    pltpu.make_async_copy(tT_hbm.at[:, pl.ds(0, 128)], stage, sem).wait()


def _extract(stage, rows_f, i, k):
    lane = jnp.full((LANES,), lax.bitwise_and(i, 127), jnp.int32)
    for m in range(H // LANES):
        j = lax.iota(jnp.int32, LANES) + (m * LANES)
        vals = plsc.load_gather(stage, [j, lane])
        rows_f[pl.ds(k * H + m * LANES, LANES)] = vals


def _sget(idx_v, k):
    base16 = (k // LANES) * LANES
    v = idx_v[pl.ds(base16, LANES)]
    m = lax.iota(jnp.int32, LANES) == (k - base16)
    return jnp.sum(jnp.where(m, v, 0))


NBUF = 8


def _gather_body(idx_hbm, tT_hbm, out_hbm, idx_v, rows_f,
                 stage0, stage1, stage2, stage3, stage4, stage5, stage6, stage7,
                 sem0, sem1, sem2, sem3, sem4, sem5, sem6, sem7):
    wid = lax.axis_index("s") * NC + lax.axis_index("c")
    base = wid * BPW
    pltpu.sync_copy(idx_hbm.at[pl.ds(base, BPW)], idx_v)

    stages = (stage0, stage1, stage2, stage3, stage4, stage5, stage6, stage7)
    sems = (sem0, sem1, sem2, sem3, sem4, sem5, sem6, sem7)
    for b in range(NBUF):
        _fetch(tT_hbm, stages[b], sems[b], _sget(idx_v, b))

    def body(g, carry):
        for b in range(NBUF):
            k = g * NBUF + b
            i = _sget(idx_v, k)
            _drain(tT_hbm, stages[b], sems[b])
            _extract(stages[b], rows_f, i, k)
            _fetch(tT_hbm, stages[b], sems[b], _sget(idx_v, k + NBUF))
        return carry

    lax.fori_loop(0, BPW // NBUF - 1, body, 0)
    for b in range(NBUF):
        k = BPW - NBUF + b
        i = _sget(idx_v, k)
        _drain(tT_hbm, stages[b], sems[b])
        _extract(stages[b], rows_f, i, k)

    pltpu.sync_copy(rows_f, out_hbm.at[pl.ds(base * H, BPW * H)])


def _sc_gather(idx, tableT):
    mesh = plsc.VectorSubcoreMesh(core_axis_name="c", subcore_axis_name="s")
    f = pl.kernel(
        _gather_body,
        out_type=jax.ShapeDtypeStruct((B * H,), jnp.float32),
        mesh=mesh,
        scratch_types=[
            pltpu.VMEM((BPW,), jnp.int32),
            pltpu.VMEM((BPW * H,), jnp.float32),
            pltpu.VMEM((H, 128), jnp.float32),
            pltpu.VMEM((H, 128), jnp.float32),
            pltpu.VMEM((H, 128), jnp.float32),
            pltpu.VMEM((H, 128), jnp.float32),
            pltpu.VMEM((H, 128), jnp.float32),
            pltpu.VMEM((H, 128), jnp.float32),
            pltpu.VMEM((H, 128), jnp.float32),
            pltpu.VMEM((H, 128), jnp.float32),
            pltpu.SemaphoreType.DMA,
            pltpu.SemaphoreType.DMA,
            pltpu.SemaphoreType.DMA,
            pltpu.SemaphoreType.DMA,
            pltpu.SemaphoreType.DMA,
            pltpu.SemaphoreType.DMA,
            pltpu.SemaphoreType.DMA,
            pltpu.SemaphoreType.DMA,
        ],
        compiler_params=pltpu.CompilerParams(
            use_tc_tiling_on_sc=True, needs_layout_passes=False),
    )
    return f(idx, tableT)


def _mlp_body(e_ref, w1_ref, b1_ref, w2t_ref, b2_ref, out_ref):
    h = jnp.dot(e_ref[...], w1_ref[...], preferred_element_type=jnp.float32)
    h = jnp.maximum(h + b1_ref[...], 0.0)
    y = jnp.sum(h * w2t_ref[...], axis=1, keepdims=True) + b2_ref[0, 0]
    out_ref[...] = y


def _mlp(e, W1, b1, W2, b2):
    return pl.pallas_call(
        _mlp_body,
        grid=(B // MLP_BLOCK,),
        in_specs=[
            pl.BlockSpec((MLP_BLOCK, H), lambda i: (i, 0)),
            pl.BlockSpec((H, H), lambda i: (0, 0)),
            pl.BlockSpec((1, H), lambda i: (0, 0)),
            pl.BlockSpec((1, H), lambda i: (0, 0)),
            pl.BlockSpec((1, 1), lambda i: (0, 0)),
        ],
        out_specs=pl.BlockSpec((MLP_BLOCK, 1), lambda i: (i, 0)),
        out_shape=jax.ShapeDtypeStruct((B, 1), jnp.float32),
    )(e, W1, b1.reshape(1, H), W2.reshape(1, H), b2.reshape(1, 1))


def kernel(users, items, table, W1, b1, W2, b2):
    idx = users.astype(jnp.int32)
    flat = _sc_gather(idx, table.T)
    e = flat.reshape(B, H)
    return _mlp(e, W1, b1, W2, b2)


# speedup vs baseline: 1.6495x; 1.0664x over previous
import jax
import jax.numpy as jnp
from jax import lax
from jax.experimental import pallas as pl
from jax.experimental.pallas import tpu as pltpu
from jax.experimental.pallas import tpu_sc as plsc

H = 64
B = 16384
NC, NS = 2, 16
NW = NC * NS
BPW = B // NW
LANES = 16

MLP_BLOCK = 2048


def _fetch(tT_hbm, stage, sem, i):
    tb = pl.multiple_of(i - lax.bitwise_and(i, 127), 128)
    for g in range(H // 8):
        pltpu.make_async_copy(
            tT_hbm.at[pl.ds(g * 8, 8), pl.ds(tb, 128)],
            stage.at[pl.ds(g * 8, 8), :],
            sem,
        ).start()


def _drain(tT_hbm, stage, sem):
    pltpu.make_async_copy(tT_hbm.at[:, pl.ds(0, 128)], stage, sem).wait()


def _extract(stage, cols_v, i, k):
    lane = jnp.full((LANES,), lax.bitwise_and(i, 127), jnp.int32)
    kvec = jnp.full((LANES,), k, jnp.int32)
    for m in range(H // LANES):
        j = lax.iota(jnp.int32, LANES) + (m * LANES)
        vals = plsc.load_gather(stage, [j, lane])
        plsc.store_scatter(cols_v, [j, kvec], vals)


def _sget(idx_v, k):
    base16 = (k // LANES) * LANES
    v = idx_v[pl.ds(base16, LANES)]
    m = lax.iota(jnp.int32, LANES) == (k - base16)
    return jnp.sum(jnp.where(m, v, 0))


NBUF = 8


def _gather_body(idx_hbm, tT_hbm, out_hbm, idx_v, cols_v,
                 stage0, stage1, stage2, stage3, stage4, stage5, stage6, stage7,
                 sem0, sem1, sem2, sem3, sem4, sem5, sem6, sem7):
    wid = lax.axis_index("s") * NC + lax.axis_index("c")
    base = wid * BPW
    pltpu.sync_copy(idx_hbm.at[pl.ds(base, BPW)], idx_v)

    stages = (stage0, stage1, stage2, stage3, stage4, stage5, stage6, stage7)
    sems = (sem0, sem1, sem2, sem3, sem4, sem5, sem6, sem7)
    for b in range(NBUF):
        _fetch(tT_hbm, stages[b], sems[b], _sget(idx_v, b))

    def body(g, carry):
        for b in range(NBUF):
            k = g * NBUF + b
            i = _sget(idx_v, k)
            _drain(tT_hbm, stages[b], sems[b])
            _extract(stages[b], cols_v, i, k)
            _fetch(tT_hbm, stages[b], sems[b], _sget(idx_v, k + NBUF))
        return carry

    lax.fori_loop(0, BPW // NBUF - 1, body, 0)
    for b in range(NBUF):
        k = BPW - NBUF + b
        i = _sget(idx_v, k)
        _drain(tT_hbm, stages[b], sems[b])
        _extract(stages[b], cols_v, i, k)

    pltpu.sync_copy(cols_v, out_hbm.at[:, pl.ds(base, BPW)])


def _sc_gather(idx, tableT):
    mesh = plsc.VectorSubcoreMesh(core_axis_name="c", subcore_axis_name="s")
    f = pl.kernel(
        _gather_body,
        out_type=jax.ShapeDtypeStruct((H, B), jnp.float32),
        mesh=mesh,
        scratch_types=[
            pltpu.VMEM((BPW,), jnp.int32),
            pltpu.VMEM((H, BPW), jnp.float32),
            pltpu.VMEM((H, 128), jnp.float32),
            pltpu.VMEM((H, 128), jnp.float32),
            pltpu.VMEM((H, 128), jnp.float32),
            pltpu.VMEM((H, 128), jnp.float32),
            pltpu.VMEM((H, 128), jnp.float32),
            pltpu.VMEM((H, 128), jnp.float32),
            pltpu.VMEM((H, 128), jnp.float32),
            pltpu.VMEM((H, 128), jnp.float32),
            pltpu.SemaphoreType.DMA,
            pltpu.SemaphoreType.DMA,
            pltpu.SemaphoreType.DMA,
            pltpu.SemaphoreType.DMA,
            pltpu.SemaphoreType.DMA,
            pltpu.SemaphoreType.DMA,
            pltpu.SemaphoreType.DMA,
            pltpu.SemaphoreType.DMA,
        ],
        compiler_params=pltpu.CompilerParams(
            use_tc_tiling_on_sc=True, needs_layout_passes=False),
    )
    return f(idx, tableT)


def _mlp_body(e_ref, w1t_ref, b1_ref, w2_ref, b2_ref, out_ref):
    h = jnp.dot(w1t_ref[...], e_ref[...], preferred_element_type=jnp.float32)
    h = jnp.maximum(h + b1_ref[...], 0.0)
    y = jnp.sum(h * w2_ref[...], axis=0, keepdims=True) + b2_ref[0, 0]
    out_ref[...] = y


def _mlp_t(eT, W1, b1, W2, b2):
    yt = pl.pallas_call(
        _mlp_body,
        grid=(B // MLP_BLOCK,),
        in_specs=[
            pl.BlockSpec((H, MLP_BLOCK), lambda i: (0, i)),
            pl.BlockSpec((H, H), lambda i: (0, 0)),
            pl.BlockSpec((H, 1), lambda i: (0, 0)),
            pl.BlockSpec((H, 1), lambda i: (0, 0)),
            pl.BlockSpec((1, 1), lambda i: (0, 0)),
        ],
        out_specs=pl.BlockSpec((1, MLP_BLOCK), lambda i: (0, i)),
        out_shape=jax.ShapeDtypeStruct((1, B), jnp.float32),
    )(eT, W1.T, b1.reshape(H, 1), W2.reshape(H, 1), b2.reshape(1, 1))
    return yt.reshape(B, 1)


def kernel(users, items, table, W1, b1, W2, b2):
    idx = users.astype(jnp.int32)
    eT = _sc_gather(idx, table.T)
    return _mlp_t(eT, W1, b1, W2, b2)
